# baseline (speedup 1.0000x reference)
import functools

import jax
import jax.numpy as jnp
from jax import lax
from jax.experimental import pallas as pl
from jax.experimental.pallas import tpu as pltpu
from jax.experimental.pallas import tpu_sc as plsc

_P = 16


def _project_body(t_ref, w_ref, b_ref, o_ref):
  o_ref[...] = (
      jnp.dot(t_ref[...].astype(jnp.bfloat16), w_ref[...],
              preferred_element_type=jnp.float32)
      + b_ref[...])


def _project_table(table2, bigw, bias):
  V8 = table2.shape[0]
  RB = 5000
  assert V8 % RB == 0
  return pl.pallas_call(
      _project_body,
      grid=(V8 // RB,),
      in_specs=[
          pl.BlockSpec((RB, 256), lambda i: (i, 0)),
          pl.BlockSpec((256, 128), lambda i: (0, 0)),
          pl.BlockSpec((1, 128), lambda i: (0, 0)),
      ],
      out_specs=pl.BlockSpec((RB, 128), lambda i: (i, 0)),
      out_shape=jax.ShapeDtypeStruct((V8, 128), jnp.float32),
  )(table2, bigw, bias)


def _pooling_kernel(B, L, V):
  mesh = plsc.VectorSubcoreMesh(core_axis_name="c", subcore_axis_name="s")
  NC, NS = mesh.num_cores, mesh.num_subcores
  NW = NC * NS
  assert B % NW == 0 and L % 8 == 0
  b_per_w = B // NW
  GROUP = 4
  CH = GROUP * L
  ngroups = b_per_w // GROUP
  assert b_per_w % GROUP == 0 and ngroups % 2 == 0
  chunks = []
  off = 0
  while off < CH:
    sz = min(128, CH - off)
    chunks.append((off, sz))
    off += sz
  assert all(o % 8 == 0 and s % 8 == 0 for o, s in chunks)

  @functools.partial(
      pl.kernel,
      out_type=jax.ShapeDtypeStruct((B, _P), jnp.float32),
      mesh=mesh,
      scratch_types=[
          pltpu.VMEM((b_per_w * L,), jnp.int32),
          pltpu.VMEM((CH, _P), jnp.float32),
          pltpu.VMEM((CH, _P), jnp.float32),
          pltpu.VMEM((b_per_w, _P), jnp.float32),
          pltpu.SemaphoreType.DMA,
          pltpu.SemaphoreType.DMA,
      ],
      compiler_params=pltpu.CompilerParams(use_tc_tiling_on_sc=False),
  )
  def kern(tp_hbm, ids_hbm, pooled_hbm, ids_v, buf0, buf1, pooled_v,
           sem0, sem1):
    wid = lax.axis_index("s") * NC + lax.axis_index("c")
    base = wid * b_per_w

    pltpu.sync_copy(ids_hbm.at[pl.ds(base * L, b_per_w * L)], ids_v)

    def fire(g, buf, sem):
      gb = g * CH
      for o, s in chunks:
        pltpu.async_copy(tp_hbm.at[ids_v.at[pl.ds(gb + o, s)]],
                         buf.at[pl.ds(o, s)], sem)

    def wait(buf, sem):
      pltpu.make_async_copy(tp_hbm.at[pl.ds(0, CH)], buf, sem).wait()

    def acc_store(i, buf, row_off):
      zero = jnp.zeros((_P,), jnp.float32)

      def body(r, carry):
        a, b, c, d = carry
        rr = row_off + r * 4
        a = a + buf[rr, :]
        b = b + buf[rr + 1, :]
        c = c + buf[rr + 2, :]
        d = d + buf[rr + 3, :]
        return (a, b, c, d)

      a, b, c, d = lax.fori_loop(0, L // 4, body, (zero,) * 4)
      pooled_v[i, :] = (a + b) + (c + d)

    def acc_group(g, buf):
      for k in range(GROUP):
        acc_store(g * GROUP + k, buf, k * L)

    fire(0, buf0, sem0)
    fire(1, buf1, sem1)

    def outer(j, carry):
      g = j * 2
      wait(buf0, sem0)
      acc_group(g, buf0)
      fire(g + 2, buf0, sem0)
      wait(buf1, sem1)
      acc_group(g + 1, buf1)
      fire(g + 3, buf1, sem1)
      return carry

    lax.fori_loop(0, ngroups // 2 - 1, outer, 0)
    wait(buf0, sem0)
    acc_group(ngroups - 2, buf0)
    wait(buf1, sem1)
    acc_group(ngroups - 1, buf1)

    pltpu.sync_copy(pooled_v, pooled_hbm.at[pl.ds(base, b_per_w)])

  return kern


def kernel(input_ids, table, W, b):
  B, L = input_ids.shape
  V, E = table.shape
  C = W.shape[0]
  ids_flat = input_ids.reshape(B * L).astype(jnp.int32)

  wpad = jnp.zeros((E, _P), jnp.float32).at[:, :C].set(W.T)
  bigw = jnp.zeros((8 * E, 8 * _P), jnp.float32)
  for k in range(8):
    bigw = lax.dynamic_update_slice(bigw, wpad, (k * E, k * _P))
  bias = jnp.tile(jnp.zeros((1, _P), jnp.float32).at[:, :C].set(b), (1, 8))
  tp8 = _project_table(table.reshape(V * E // 256, 256),
                       bigw.astype(jnp.bfloat16), bias)
  tp = tp8.reshape(V, _P)

  t2 = table.reshape(V * E // 256, 256)
  def _copy_body(t_ref, o_ref):
    o_ref[...] = t_ref[...]
  cp = pl.pallas_call(
      _copy_body,
      grid=(t2.shape[0] // 5000,),
      in_specs=[pl.BlockSpec((5000, 256), lambda i: (i, 0))],
      out_specs=pl.BlockSpec((5000, 256), lambda i: (i, 0)),
      out_shape=jax.ShapeDtypeStruct(t2.shape, jnp.float32),
  )(t2)
  return cp[:B, :C]
  pooled = _pooling_kernel(B, L, V)(tp, ids_flat)
  return pooled[:, :C]


# speedup vs baseline: 1.0575x; 1.0575x over previous
import functools

import jax
import jax.numpy as jnp
from jax import lax
from jax.experimental import pallas as pl
from jax.experimental.pallas import tpu as pltpu
from jax.experimental.pallas import tpu_sc as plsc


def _pooling_kernel(B, L, V, E, group):
  mesh = plsc.VectorSubcoreMesh(core_axis_name="c", subcore_axis_name="s")
  NC, NS = mesh.num_cores, mesh.num_subcores
  NW = NC * NS
  assert B % NW == 0 and E == 32 and L % 8 == 0
  b_per_w = B // NW
  GROUP = group
  CH = GROUP * L
  ngroups = b_per_w // GROUP
  assert b_per_w % GROUP == 0 and ngroups % 2 == 0
  chunks = []
  off = 0
  while off < CH:
    sz = min(128, CH - off)
    chunks.append((off, sz))
    off += sz
  assert all(o % 8 == 0 and s % 8 == 0 for o, s in chunks)

  @functools.partial(
      pl.kernel,
      out_type=jax.ShapeDtypeStruct((B, E), jnp.float32),
      mesh=mesh,
      scratch_types=[
          pltpu.VMEM((b_per_w * L,), jnp.int32),
          pltpu.VMEM((CH, E), jnp.float32),
          pltpu.VMEM((CH, E), jnp.float32),
          pltpu.VMEM((b_per_w, E), jnp.float32),
          pltpu.SemaphoreType.DMA,
          pltpu.SemaphoreType.DMA,
      ],
      compiler_params=pltpu.CompilerParams(use_tc_tiling_on_sc=False),
  )
  def kern(table_hbm, ids_hbm, pooled_hbm, ids_v, buf0, buf1, pooled_v,
           sem0, sem1):
    wid = lax.axis_index("s") * NC + lax.axis_index("c")
    base = wid * b_per_w

    pltpu.sync_copy(ids_hbm.at[pl.ds(base * L, b_per_w * L)], ids_v)

    def fire(g, buf, sem):
      gb = g * CH
      for o, s in chunks:
        pltpu.async_copy(table_hbm.at[ids_v.at[pl.ds(gb + o, s)]],
                         buf.at[pl.ds(o, s)], sem)

    def wait(buf, sem):
      pltpu.make_async_copy(table_hbm.at[pl.ds(0, CH)], buf, sem).wait()

    h = E // 2

    def acc_store(i, buf, row_off):
      zero = jnp.zeros((h,), jnp.float32)

      def body(r, carry):
        a0, a1, b0, b1, c0, c1, d0, d1 = carry
        rr = row_off + r * 4
        a0 = a0 + buf[rr, pl.ds(0, h)]
        a1 = a1 + buf[rr, pl.ds(h, h)]
        b0 = b0 + buf[rr + 1, pl.ds(0, h)]
        b1 = b1 + buf[rr + 1, pl.ds(h, h)]
        c0 = c0 + buf[rr + 2, pl.ds(0, h)]
        c1 = c1 + buf[rr + 2, pl.ds(h, h)]
        d0 = d0 + buf[rr + 3, pl.ds(0, h)]
        d1 = d1 + buf[rr + 3, pl.ds(h, h)]
        return (a0, a1, b0, b1, c0, c1, d0, d1)

      a0, a1, b0, b1, c0, c1, d0, d1 = lax.fori_loop(
          0, L // 4, body, (zero,) * 8)
      pooled_v[i, pl.ds(0, h)] = (a0 + b0) + (c0 + d0)
      pooled_v[i, pl.ds(h, h)] = (a1 + b1) + (c1 + d1)

    def acc_group(g, buf):
      for k in range(GROUP):
        acc_store(g * GROUP + k, buf, k * L)

    fire(0, buf0, sem0)
    fire(1, buf1, sem1)

    def outer(j, carry):
      g = j * 2
      wait(buf0, sem0)
      acc_group(g, buf0)
      fire(g + 2, buf0, sem0)
      wait(buf1, sem1)
      acc_group(g + 1, buf1)
      fire(g + 3, buf1, sem1)
      return carry

    lax.fori_loop(0, ngroups // 2 - 1, outer, 0)
    wait(buf0, sem0)
    acc_group(ngroups - 2, buf0)
    wait(buf1, sem1)
    acc_group(ngroups - 1, buf1)

    pltpu.sync_copy(pooled_v, pooled_hbm.at[pl.ds(base, b_per_w)])

  return kern


def _proj_body(p_ref, wt_ref, b_ref, o_ref):
  o_ref[...] = (
      jnp.dot(p_ref[...], wt_ref[...], preferred_element_type=jnp.float32)
      + b_ref[...])


def kernel(input_ids, table, W, b):
  B, L = input_ids.shape
  V, E = table.shape
  C = W.shape[0]
  ids_flat = input_ids.reshape(B * L).astype(jnp.int32)

  pooled = _pooling_kernel(B, L, V, E, 4)(table, ids_flat)

  wt = W.T.astype(jnp.float32)
  bias = (jnp.float32(L) * b).reshape(1, C)
  out = pl.pallas_call(
      _proj_body,
      out_shape=jax.ShapeDtypeStruct((B, C), jnp.float32),
  )(pooled, wt, bias)
  return out
